# baseline (speedup 1.0000x reference)
import jax
import jax.numpy as jnp
from jax.experimental import pallas as pl
from jax.experimental.pallas import tpu as pltpu

_N_STATES = 16
_HEAD = 128


_ROW_STRIPES = 8
_COL_STRIPES = 4


def _gibbs_body(x_any, w16_ref, g_ref, o_any, xh_scr, head_scr,
                sem_b, sem_x, sem_h):
    n_chains = g_ref.shape[0]
    flat = x_any.shape[1]
    rc = n_chains // _ROW_STRIPES
    col_edges = [_HEAD] + [flat * (k + 1) // _COL_STRIPES
                           for k in range(_COL_STRIPES)]
    bulk = []
    for r in range(_ROW_STRIPES):
        for c in range(_COL_STRIPES):
            c0, c1 = col_edges[c], col_edges[c + 1]
            dma = pltpu.make_async_copy(
                x_any.at[pl.ds(r * rc, rc), pl.ds(c0, c1 - c0)],
                o_any.at[pl.ds(r * rc, rc), pl.ds(c0, c1 - c0)],
                sem_b.at[r, c])
            dma.start()
            bulk.append(dma)
    xh_dma = pltpu.make_async_copy(x_any.at[:, :_HEAD], xh_scr, sem_x)
    xh_dma.start()
    logits = w16_ref[...] + g_ref[...]
    m = jnp.max(logits, axis=1, keepdims=True)
    iota_s = jax.lax.broadcasted_iota(jnp.int32, (n_chains, _N_STATES), 1)
    sel = jnp.min(jnp.where(logits == m, iota_s, _N_STATES), axis=1,
                  keepdims=True)
    lane = jax.lax.broadcasted_iota(jnp.int32, (n_chains, _HEAD), 1)
    onehot = (lane == sel).astype(g_ref.dtype)
    xh_dma.wait()
    head_scr[...] = jnp.where(lane < _N_STATES, onehot, xh_scr[...])
    head = pltpu.make_async_copy(head_scr, o_any.at[:, :_HEAD], sem_h)
    head.start()
    for dma in bulk:
        dma.wait()
    head.wait()


def kernel(x, W):
    n_chains, n_dims, n_states = x.shape
    flat = n_dims * n_states
    x2 = x.reshape(n_chains, flat)
    w16 = W[:n_states].reshape(1, n_states)
    g = jax.random.gumbel(jax.random.key(42), (n_chains, n_states),
                          dtype=x.dtype)
    out = pl.pallas_call(
        _gibbs_body,
        in_specs=[
            pl.BlockSpec(memory_space=pltpu.MemorySpace.HBM),
            pl.BlockSpec(memory_space=pltpu.MemorySpace.VMEM),
            pl.BlockSpec(memory_space=pltpu.MemorySpace.VMEM),
        ],
        out_specs=pl.BlockSpec(memory_space=pltpu.MemorySpace.HBM),
        out_shape=jax.ShapeDtypeStruct((n_chains, flat), x.dtype),
        scratch_shapes=[
            pltpu.VMEM((n_chains, _HEAD), x.dtype),
            pltpu.VMEM((n_chains, _HEAD), x.dtype),
            pltpu.SemaphoreType.DMA((_ROW_STRIPES, _COL_STRIPES)),
            pltpu.SemaphoreType.DMA,
            pltpu.SemaphoreType.DMA,
        ],
    )(x2, w16, g)
    return out.reshape(n_chains, n_dims, n_states)


# speedup vs baseline: 4.3983x; 4.3983x over previous
import jax
import jax.numpy as jnp
from jax.experimental import pallas as pl
from jax.experimental.pallas import tpu as pltpu

_N_STATES = 16
_CB = 8


def _gibbs_body(x_ref, w16_ref, g_ref, o_ref):
    xv = x_ref[...]
    logits = w16_ref[...] + g_ref[...]
    m = jnp.max(logits, axis=1, keepdims=True)
    iota = jax.lax.broadcasted_iota(jnp.int32, (_CB, _N_STATES), 1)
    sel = jnp.min(jnp.where(logits == m, iota, _N_STATES), axis=1,
                  keepdims=True)
    o_ref[...] = xv
    o_ref[:, :_N_STATES] = (iota == sel).astype(xv.dtype)


def kernel(x, W):
    n_chains, n_dims, n_states = x.shape
    flat = n_dims * n_states
    x2 = x.reshape(n_chains, flat)
    w16 = W[:n_states].reshape(1, n_states)
    g = jax.random.gumbel(jax.random.key(42), (n_chains, n_states),
                          dtype=x.dtype)
    out = pl.pallas_call(
        _gibbs_body,
        grid=(n_chains // _CB,),
        in_specs=[
            pl.BlockSpec((_CB, flat), lambda i: (i, 0)),
            pl.BlockSpec((1, n_states), lambda i: (0, 0)),
            pl.BlockSpec((_CB, n_states), lambda i: (i, 0)),
        ],
        out_specs=pl.BlockSpec((_CB, flat), lambda i: (i, 0)),
        out_shape=jax.ShapeDtypeStruct((n_chains, flat), x.dtype),
    )(x2, w16, g)
    return out.reshape(n_chains, n_dims, n_states)
